# baseline (speedup 1.0000x reference)
import functools

import jax
import jax.numpy as jnp
import numpy as np
from jax import lax
from jax.experimental import pallas as pl
from jax.experimental.pallas import tpu as pltpu
from jax.experimental.pallas import tpu_sc as plsc

_BATCH = 128
_F = 2048
_OUT = 2048
_CMS = 32
_U = 64
_TILE = 1024
_GRID = _OUT // _TILE

_ROWS = _BATCH * _CMS
_NW = 32
_RPW = _ROWS // _NW
_WORDS = _RPW * _U


def _gumbel_traced():
    base = jax.random.key(42)
    keys = jax.vmap(jax.random.fold_in, in_axes=(None, 0))(base, jnp.arange(_CMS))
    return jax.vmap(lambda k: jax.random.gumbel(k, (_BATCH, _U), jnp.float32))(keys)


_NOISE = None


def _noise():
    global _NOISE
    if _NOISE is None:
        with jax.ensure_compile_time_eval():
            _NOISE = np.asarray(_gumbel_traced())
    return _NOISE


def _tc_body(x_ref, w_ref, b_ref, g_ref, out_ref, xt_ref, zs_ref):
    k = pl.program_id(0)

    @pl.when(k == 0)
    def _stage():
        xt_ref[...] = jnp.transpose(x_ref[...])

    zt = jax.lax.dot_general(
        w_ref[...], xt_ref[...], (((1,), (0,)), ((), ())),
        preferred_element_type=jnp.float32)
    na = jnp.sum(xt_ref[...], axis=0, keepdims=True)
    zs_ref[pl.ds(k * _TILE, _TILE), :] = (zt + b_ref[...]) / na

    @pl.when(k == _GRID - 1)
    def _scores():
        z = zs_ref[...]
        m = jnp.max(z, axis=0, keepdims=True)
        row_cm = jax.lax.broadcasted_iota(jnp.int32, (_OUT, _BATCH), 0) // _U
        lane = jax.lax.broadcasted_iota(jnp.int32, (_OUT, _BATCH), 1)
        sel = (row_cm == lane).astype(jnp.float32)
        mrows = jax.lax.dot_general(
            sel, m, (((1,), (1,)), ((), ())),
            preferred_element_type=jnp.float32)
        gt = jnp.transpose(g_ref[...], (0, 2, 1))
        s3 = jnp.exp(z - mrows).reshape(_CMS, _U, _BATCH) + gt
        out_ref[...] = jnp.transpose(s3.reshape(_OUT, _BATCH))


def _sc_body(scores_hbm, out_hbm, buf, obuf, sem):
    wid = lax.axis_index("s") * 2 + lax.axis_index("c")
    base = wid * _WORDS
    pltpu.sync_copy(scores_hbm.at[pl.ds(base, _WORDS)], buf)

    riota = lax.iota(jnp.int32, 16) * _U

    def zero_step(i, c):
        obuf[pl.ds(i * 16, 16)] = jnp.zeros((16,), jnp.float32)
        return c

    lax.fori_loop(0, _WORDS // 16, zero_step, 0)

    def group(g, c):
        gbase = g * 16 * _U
        mx0 = jnp.full((16,), -jnp.inf, jnp.float32)
        am0 = jnp.zeros((16,), jnp.int32)

        def unit(u, carry):
            mx, am = carry
            vals = plsc.load_gather(buf, [gbase + riota + u])
            upd = vals > mx
            return (jnp.where(upd, vals, mx), jnp.where(upd, u, am))

        mx, am = lax.fori_loop(0, _U, unit, (mx0, am0))
        plsc.store_scatter(obuf, [gbase + riota + am],
                           jnp.ones((16,), jnp.float32))
        return c

    lax.fori_loop(0, _RPW // 16, group, 0)
    pltpu.sync_copy(obuf, out_hbm.at[pl.ds(base, _WORDS)])


def _sc_sample(scores_flat):
    mesh = plsc.VectorSubcoreMesh(core_axis_name="c", subcore_axis_name="s",
                                  num_cores=2, num_subcores=16)
    k = functools.partial(
        pl.kernel,
        mesh=mesh,
        compiler_params=pltpu.CompilerParams(needs_layout_passes=False),
        out_type=jax.ShapeDtypeStruct((_ROWS * _U,), jnp.float32),
        scratch_types=[
            pltpu.VMEM((_WORDS,), jnp.float32),
            pltpu.VMEM((_WORDS,), jnp.float32),
            pltpu.SemaphoreType.DMA,
        ],
    )(_sc_body)
    return k(scores_flat)


@jax.jit
def _impl(x, W_in, b_in):
    b2 = b_in.reshape(_OUT, 1)
    g = jnp.asarray(_noise())
    scores = pl.pallas_call(
        _tc_body,
        grid=(_GRID,),
        in_specs=[
            pl.BlockSpec((_BATCH, _F), lambda k: (0, 0)),
            pl.BlockSpec((_TILE, _F), lambda k: (k, 0)),
            pl.BlockSpec((_TILE, 1), lambda k: (k, 0)),
            pl.BlockSpec((_CMS, _BATCH, _U), lambda k: (0, 0, 0)),
        ],
        out_specs=pl.BlockSpec((_BATCH, _OUT), lambda k: (0, 0)),
        out_shape=jax.ShapeDtypeStruct((_BATCH, _OUT), jnp.float32),
        scratch_shapes=[
            pltpu.VMEM((_F, _BATCH), jnp.float32),
            pltpu.VMEM((_OUT, _BATCH), jnp.float32),
        ],
    )(x, W_in, b2, g)
    onehot = _sc_sample(scores.reshape(-1))
    return onehot.reshape(_BATCH, _OUT)


def kernel(x, W_in, b_in):
    return _impl(x, W_in, b_in)


# speedup vs baseline: 1.0934x; 1.0934x over previous
import functools

import jax
import jax.numpy as jnp
import numpy as np
from jax import lax
from jax.experimental import pallas as pl
from jax.experimental.pallas import tpu as pltpu
from jax.experimental.pallas import tpu_sc as plsc

_BATCH = 128
_F = 2048
_OUT = 2048
_CMS = 32
_U = 64
_TILE = 1024
_GRID = _OUT // _TILE

_ROWS = _BATCH * _CMS
_NW = 32
_RPW = _ROWS // _NW
_WORDS = _RPW * _U


def _gumbel_traced():
    base = jax.random.key(42)
    keys = jax.vmap(jax.random.fold_in, in_axes=(None, 0))(base, jnp.arange(_CMS))
    return jax.vmap(lambda k: jax.random.gumbel(k, (_BATCH, _U), jnp.float32))(keys)


_NOISE = None


def _noise():
    global _NOISE
    if _NOISE is None:
        with jax.ensure_compile_time_eval():
            _NOISE = np.asarray(_gumbel_traced())
    return _NOISE


def _tc_body(x_ref, w_ref, b_ref, g_ref, out_ref, xt_ref, zs_ref):
    k = pl.program_id(0)

    @pl.when(k == 0)
    def _stage():
        xt_ref[...] = jnp.transpose(x_ref[...])

    zt = jax.lax.dot_general(
        w_ref[...], xt_ref[...], (((1,), (0,)), ((), ())),
        preferred_element_type=jnp.float32)
    na = jnp.sum(xt_ref[...], axis=0, keepdims=True)
    zs_ref[pl.ds(k * _TILE, _TILE), :] = (zt + b_ref[...]) / na

    @pl.when(k == _GRID - 1)
    def _scores():
        z = zs_ref[...]
        m = jnp.max(z, axis=0, keepdims=True)
        row_cm = jax.lax.broadcasted_iota(jnp.int32, (_OUT, _BATCH), 0) // _U
        lane = jax.lax.broadcasted_iota(jnp.int32, (_OUT, _BATCH), 1)
        sel = (row_cm == lane).astype(jnp.float32)
        mrows = jax.lax.dot_general(
            sel, m, (((1,), (1,)), ((), ())),
            preferred_element_type=jnp.float32)
        gt = jnp.transpose(g_ref[...], (0, 2, 1))
        s3 = jnp.exp(z - mrows).reshape(_CMS, _U, _BATCH) + gt
        out_ref[...] = jnp.transpose(s3.reshape(_OUT, _BATCH))


def _sc_body(scores_hbm, out_hbm, buf, obuf, sem):
    wid = lax.axis_index("s") * 2 + lax.axis_index("c")
    base = wid * _WORDS
    pltpu.sync_copy(scores_hbm.at[pl.ds(base, _WORDS)], buf)

    riota = lax.iota(jnp.int32, 16) * _U
    zeros16 = jnp.zeros((16,), jnp.float32)

    def zero_step(i, c):
        b = i * 256
        for j in range(16):
            obuf[pl.ds(b + j * 16, 16)] = zeros16
        return c

    lax.fori_loop(0, _WORDS // 256, zero_step, 0)

    def group(g, c):
        base_idx = g * (16 * _U) + riota
        mx = jnp.full((16,), -jnp.inf, jnp.float32)
        am = jnp.zeros((16,), jnp.int32)
        for u in range(_U):
            vals = plsc.load_gather(buf, [base_idx + u])
            upd = vals > mx
            mx = jnp.where(upd, vals, mx)
            am = jnp.where(upd, u, am)
        plsc.store_scatter(obuf, [base_idx + am],
                           jnp.ones((16,), jnp.float32))
        return c

    lax.fori_loop(0, _RPW // 16, group, 0)
    pltpu.sync_copy(obuf, out_hbm.at[pl.ds(base, _WORDS)])


def _sc_sample(scores_flat):
    mesh = plsc.VectorSubcoreMesh(core_axis_name="c", subcore_axis_name="s",
                                  num_cores=2, num_subcores=16)
    k = functools.partial(
        pl.kernel,
        mesh=mesh,
        compiler_params=pltpu.CompilerParams(needs_layout_passes=False),
        out_type=jax.ShapeDtypeStruct((_ROWS * _U,), jnp.float32),
        scratch_types=[
            pltpu.VMEM((_WORDS,), jnp.float32),
            pltpu.VMEM((_WORDS,), jnp.float32),
            pltpu.SemaphoreType.DMA,
        ],
    )(_sc_body)
    return k(scores_flat)


@jax.jit
def _impl(x, W_in, b_in):
    b2 = b_in.reshape(_OUT, 1)
    g = jnp.asarray(_noise())
    scores = pl.pallas_call(
        _tc_body,
        grid=(_GRID,),
        in_specs=[
            pl.BlockSpec((_BATCH, _F), lambda k: (0, 0)),
            pl.BlockSpec((_TILE, _F), lambda k: (k, 0)),
            pl.BlockSpec((_TILE, 1), lambda k: (k, 0)),
            pl.BlockSpec((_CMS, _BATCH, _U), lambda k: (0, 0, 0)),
        ],
        out_specs=pl.BlockSpec((_BATCH, _OUT), lambda k: (0, 0)),
        out_shape=jax.ShapeDtypeStruct((_BATCH, _OUT), jnp.float32),
        scratch_shapes=[
            pltpu.VMEM((_F, _BATCH), jnp.float32),
            pltpu.VMEM((_OUT, _BATCH), jnp.float32),
        ],
    )(x, W_in, b2, g)
    onehot = _sc_sample(scores.reshape(-1))
    return onehot.reshape(_BATCH, _OUT)


def kernel(x, W_in, b_in):
    return _impl(x, W_in, b_in)


# speedup vs baseline: 3.0974x; 2.8327x over previous
import jax
import jax.numpy as jnp
import numpy as np
from jax.experimental import pallas as pl
from jax.experimental.pallas import tpu as pltpu

_BATCH = 128
_F = 2048
_OUT = 2048
_CMS = 32
_U = 64
_TILE = 1024
_GRID = _OUT // _TILE


def _gumbel_traced():
    base = jax.random.key(42)
    keys = jax.vmap(jax.random.fold_in, in_axes=(None, 0))(base, jnp.arange(_CMS))
    return jax.vmap(lambda k: jax.random.gumbel(k, (_BATCH, _U), jnp.float32))(keys)


_NOISE = None


def _noise():
    global _NOISE
    if _NOISE is None:
        with jax.ensure_compile_time_eval():
            _NOISE = np.asarray(_gumbel_traced())
    return _NOISE


def _body(x_ref, w_ref, b_ref, g_ref, out_ref, xt_ref, zs_ref):
    k = pl.program_id(0)

    @pl.when(k == 0)
    def _stage():
        xt_ref[...] = jnp.transpose(x_ref[...])

    zt = jax.lax.dot_general(
        w_ref[...], xt_ref[...], (((1,), (0,)), ((), ())),
        preferred_element_type=jnp.float32)
    na = jnp.sum(xt_ref[...], axis=0, keepdims=True)
    zs_ref[pl.ds(k * _TILE, _TILE), :] = (zt + b_ref[...]) / na

    @pl.when(k == _GRID - 1)
    def _sample():
        z = zs_ref[...]
        m = jnp.max(z, axis=0, keepdims=True)
        row_cm = jax.lax.broadcasted_iota(jnp.int32, (_OUT, _BATCH), 0) // _U
        lane = jax.lax.broadcasted_iota(jnp.int32, (_OUT, _BATCH), 1)
        sel = (row_cm == lane).astype(jnp.float32)
        mrows = jax.lax.dot_general(
            sel, m, (((1,), (1,)), ((), ())),
            preferred_element_type=jnp.float32)
        gt = jnp.transpose(g_ref[...], (0, 2, 1))
        s3 = jnp.exp(z - mrows).reshape(_CMS, _U, _BATCH) + gt
        gm = jnp.max(s3, axis=1, keepdims=True)
        sub = jax.lax.broadcasted_iota(jnp.int32, (_CMS, _U, _BATCH), 1)
        idx = jnp.min(jnp.where(s3 >= gm, sub, _U), axis=1)
        idx_t = jnp.transpose(idx.astype(jnp.float32))
        cm_of_col = jax.lax.broadcasted_iota(jnp.int32, (_CMS, _OUT), 1) // _U
        cm_row = jax.lax.broadcasted_iota(jnp.int32, (_CMS, _OUT), 0)
        selc = (cm_of_col == cm_row).astype(jnp.float32)
        chosen = jax.lax.dot_general(
            idx_t, selc, (((1,), (0,)), ((), ())),
            preferred_element_type=jnp.float32)
        unit = (jax.lax.broadcasted_iota(jnp.int32, (_BATCH, _OUT), 1) % _U
                ).astype(jnp.float32)
        out_ref[...] = (unit == chosen).astype(jnp.float32)


@jax.jit
def _impl(x, W_in, b_in):
    b2 = b_in.reshape(_OUT, 1)
    g = jnp.asarray(_noise())
    return pl.pallas_call(
        _body,
        grid=(_GRID,),
        in_specs=[
            pl.BlockSpec((_BATCH, _F), lambda k: (0, 0)),
            pl.BlockSpec((_TILE, _F), lambda k: (k, 0)),
            pl.BlockSpec((_TILE, 1), lambda k: (k, 0)),
            pl.BlockSpec((_CMS, _BATCH, _U), lambda k: (0, 0, 0)),
        ],
        out_specs=pl.BlockSpec((_BATCH, _OUT), lambda k: (0, 0)),
        out_shape=jax.ShapeDtypeStruct((_BATCH, _OUT), jnp.float32),
        scratch_shapes=[
            pltpu.VMEM((_F, _BATCH), jnp.float32),
            pltpu.VMEM((_OUT, _BATCH), jnp.float32),
        ],
    )(x, W_in, b2, g)


def kernel(x, W_in, b_in):
    return _impl(x, W_in, b_in)
